# baseline (speedup 1.0000x reference)
import functools

import jax
import jax.numpy as jnp
from jax import lax
from jax.experimental import pallas as pl
from jax.experimental.pallas import tpu as pltpu
from jax.experimental.pallas import tpu_sc as plsc

VOCAB = 1000000
DIM = 32
HALF = 16
NT = 5
NW_CTX = 10

NUM_CORES = 2
NUM_SUBCORES = 16
NUM_WORKERS = NUM_CORES * NUM_SUBCORES

BATCH = 16384
B_PER_W = BATCH // NUM_WORKERS
CHUNK = 128
NCHUNK = B_PER_W // CHUNK
G_CTX = CHUNK * NW_CTX // 128
G_TGT = CHUNK * NT // 128

_mesh = plsc.VectorSubcoreMesh(core_axis_name="c", subcore_axis_name="s")


@functools.partial(
    pl.kernel,
    mesh=_mesh,
    compiler_params=pltpu.CompilerParams(
        needs_layout_passes=False, use_tc_tiling_on_sc=False),
    out_type=jax.ShapeDtypeStruct((BATCH * NT,), jnp.float32),
    scratch_types=[
        pltpu.VMEM((B_PER_W * NW_CTX,), jnp.int32),
        pltpu.VMEM((B_PER_W * NT,), jnp.int32),
        pltpu.VMEM((CHUNK * NW_CTX, DIM), jnp.bfloat16),
        pltpu.VMEM((CHUNK * NT, DIM), jnp.bfloat16),
        pltpu.VMEM((CHUNK * NT,), jnp.float32),
        pltpu.SemaphoreType.DMA,
    ],
)
def _cbow_sc(ctx_idx, tgt_idx, ctx_table, tgt_table, out,
             cidx, tidx, crows, trows, outv, sem):
    wid = lax.axis_index("s") * NUM_CORES + lax.axis_index("c")

    pltpu.sync_copy(ctx_idx.at[pl.ds(wid * B_PER_W * NW_CTX,
                                     B_PER_W * NW_CTX)], cidx)
    pltpu.sync_copy(tgt_idx.at[pl.ds(wid * B_PER_W * NT, B_PER_W * NT)], tidx)

    def chunk_body(k, carry):
        def c_slice(g):
            return cidx.at[pl.ds(k * CHUNK * NW_CTX + g * 128, 128)]

        def t_slice(g):
            return tidx.at[pl.ds(k * CHUNK * NT + g * 128, 128)]

        for g in range(G_CTX):
            pltpu.async_copy(ctx_table.at[c_slice(g)],
                             crows.at[pl.ds(g * 128, 128)], sem)
        for g in range(G_TGT):
            pltpu.async_copy(tgt_table.at[t_slice(g)],
                             trows.at[pl.ds(g * 128, 128)], sem)
        for g in range(G_CTX):
            pltpu.make_async_copy(ctx_table.at[c_slice(g)],
                                  crows.at[pl.ds(g * 128, 128)], sem).wait()
        for g in range(G_TGT):
            pltpu.make_async_copy(tgt_table.at[t_slice(g)],
                                  trows.at[pl.ds(g * 128, 128)], sem).wait()

        last_lane = lax.iota(jnp.int32, HALF) == (HALF - 1)

        def elem(b, c2):
            r = b * NW_CTX
            s0, s1 = plsc.unpack(crows[r, pl.ds(0, DIM)],
                                 format=plsc.PackFormat.INTERLEAVED)
            for w in range(1, NW_CTX):
                e0, e1 = plsc.unpack(crows[r + w, pl.ds(0, DIM)],
                                     format=plsc.PackFormat.INTERLEAVED)
                s0 = s0 + e0
                s1 = s1 + e1
            t = b * NT
            for n in range(NT):
                t0, t1 = plsc.unpack(trows[t + n, pl.ds(0, DIM)],
                                     format=plsc.PackFormat.INTERLEAVED)
                q = t0 * s0 + t1 * s1
                c = plsc.cumsum(q)
                plsc.store_scatter(outv, [jnp.full((HALF,), t + n, jnp.int32)],
                                   c, mask=last_lane)
            return c2

        lax.fori_loop(0, CHUNK, elem, 0)
        pltpu.sync_copy(outv,
                        out.at[pl.ds(wid * (B_PER_W * NT) + k * CHUNK * NT,
                                     CHUNK * NT)])
        return carry

    lax.fori_loop(0, NCHUNK, chunk_body, 0)


def kernel(context, target, ctx_table, tgt_table):
    ctx_idx = context.astype(jnp.int32).reshape(-1)
    tgt_idx = target.astype(jnp.int32).reshape(-1)
    out = _cbow_sc(ctx_idx, tgt_idx, ctx_table.astype(jnp.bfloat16),
                   tgt_table.astype(jnp.bfloat16))
    return out.reshape(BATCH, NT)


# speedup vs baseline: 1.1633x; 1.1633x over previous
import functools

import jax
import jax.numpy as jnp
from jax import lax
from jax.experimental import pallas as pl
from jax.experimental.pallas import tpu as pltpu
from jax.experimental.pallas import tpu_sc as plsc

VOCAB = 1000000
DIM = 32
HALF = 16
NT = 5
NW_CTX = 10

NUM_CORES = 2
NUM_SUBCORES = 16
NUM_WORKERS = NUM_CORES * NUM_SUBCORES

BATCH = 16384
B_PER_W = BATCH // NUM_WORKERS
CHUNK = 128
NCHUNK = B_PER_W // CHUNK
G_CTX = CHUNK * NW_CTX // 128
G_TGT = CHUNK * NT // 128

_mesh = plsc.VectorSubcoreMesh(core_axis_name="c", subcore_axis_name="s")


@functools.partial(
    pl.kernel,
    mesh=_mesh,
    compiler_params=pltpu.CompilerParams(
        needs_layout_passes=False, use_tc_tiling_on_sc=False),
    out_type=jax.ShapeDtypeStruct((BATCH * NT,), jnp.float32),
    scratch_types=[
        pltpu.VMEM((B_PER_W * NW_CTX,), jnp.int32),
        pltpu.VMEM((B_PER_W * NT,), jnp.int32),
        pltpu.VMEM((CHUNK * NW_CTX, DIM), jnp.float32),
        pltpu.VMEM((CHUNK * NT, DIM), jnp.float32),
        pltpu.VMEM((CHUNK * NT,), jnp.float32),
        pltpu.SemaphoreType.DMA,
    ],
)
def _cbow_sc(ctx_idx, tgt_idx, ctx_table, tgt_table, out,
             cidx, tidx, crows, trows, outv, sem):
    wid = lax.axis_index("s") * NUM_CORES + lax.axis_index("c")

    pltpu.sync_copy(ctx_idx.at[pl.ds(wid * B_PER_W * NW_CTX,
                                     B_PER_W * NW_CTX)], cidx)
    pltpu.sync_copy(tgt_idx.at[pl.ds(wid * B_PER_W * NT, B_PER_W * NT)], tidx)

    def chunk_body(k, carry):
        def c_slice(g):
            return cidx.at[pl.ds(k * CHUNK * NW_CTX + g * 128, 128)]

        def t_slice(g):
            return tidx.at[pl.ds(k * CHUNK * NT + g * 128, 128)]

        for g in range(G_CTX):
            pltpu.async_copy(ctx_table.at[c_slice(g)],
                             crows.at[pl.ds(g * 128, 128)], sem)
        for g in range(G_TGT):
            pltpu.async_copy(tgt_table.at[t_slice(g)],
                             trows.at[pl.ds(g * 128, 128)], sem)
        for g in range(G_CTX):
            pltpu.make_async_copy(ctx_table.at[c_slice(g)],
                                  crows.at[pl.ds(g * 128, 128)], sem).wait()
        for g in range(G_TGT):
            pltpu.make_async_copy(tgt_table.at[t_slice(g)],
                                  trows.at[pl.ds(g * 128, 128)], sem).wait()

        last_lane = lax.iota(jnp.int32, HALF) == (HALF - 1)

        def elem(b, c2):
            r = b * NW_CTX
            s0 = crows[r, pl.ds(0, HALF)]
            s1 = crows[r, pl.ds(HALF, HALF)]
            for w in range(1, NW_CTX):
                s0 = s0 + crows[r + w, pl.ds(0, HALF)]
                s1 = s1 + crows[r + w, pl.ds(HALF, HALF)]
            t = b * NT
            for n in range(NT):
                q = (trows[t + n, pl.ds(0, HALF)] * s0
                     + trows[t + n, pl.ds(HALF, HALF)] * s1)
                c = plsc.cumsum(q)
                plsc.store_scatter(outv, [jnp.full((HALF,), t + n, jnp.int32)],
                                   c, mask=last_lane)
            return c2

        lax.fori_loop(0, CHUNK, elem, 0)
        pltpu.sync_copy(outv,
                        out.at[pl.ds(wid * (B_PER_W * NT) + k * CHUNK * NT,
                                     CHUNK * NT)])
        return carry

    lax.fori_loop(0, NCHUNK, chunk_body, 0)


def kernel(context, target, ctx_table, tgt_table):
    ctx_idx = context.astype(jnp.int32).reshape(-1)
    tgt_idx = target.astype(jnp.int32).reshape(-1)
    out = _cbow_sc(ctx_idx, tgt_idx, ctx_table, tgt_table)
    return out.reshape(BATCH, NT)


# speedup vs baseline: 1.1746x; 1.0096x over previous
import functools

import jax
import jax.numpy as jnp
from jax import lax
from jax.experimental import pallas as pl
from jax.experimental.pallas import tpu as pltpu
from jax.experimental.pallas import tpu_sc as plsc

VOCAB = 1000000
DIM = 32
HALF = 16
NT = 5
NW_CTX = 10

NUM_CORES = 2
NUM_SUBCORES = 16
NUM_WORKERS = NUM_CORES * NUM_SUBCORES

BATCH = 16384
B_PER_W = BATCH // NUM_WORKERS
CHUNK = 64
NCHUNK = B_PER_W // CHUNK
G_CTX = CHUNK * NW_CTX // 128
G_TGT = 5
TS = CHUNK * NT // G_TGT

_mesh = plsc.VectorSubcoreMesh(core_axis_name="c", subcore_axis_name="s")


@functools.partial(
    pl.kernel,
    mesh=_mesh,
    compiler_params=pltpu.CompilerParams(
        needs_layout_passes=False, use_tc_tiling_on_sc=False),
    out_type=jax.ShapeDtypeStruct((BATCH * NT,), jnp.float32),
    scratch_types=[
        pltpu.VMEM((B_PER_W * NW_CTX,), jnp.int32),
        pltpu.VMEM((B_PER_W * NT,), jnp.int32),
        pltpu.VMEM((CHUNK * NW_CTX, DIM), jnp.float32),
        pltpu.VMEM((CHUNK * NW_CTX, DIM), jnp.float32),
        pltpu.VMEM((CHUNK * NT, DIM), jnp.float32),
        pltpu.VMEM((CHUNK * NT, DIM), jnp.float32),
        pltpu.VMEM((CHUNK * NT,), jnp.float32),
        pltpu.SemaphoreType.DMA,
        pltpu.SemaphoreType.DMA,
    ],
)
def _cbow_sc(ctx_idx, tgt_idx, ctx_table, tgt_table, out,
             cidx, tidx, crows_a, crows_b, trows_a, trows_b, outv,
             sem_a, sem_b):
    wid = lax.axis_index("s") * NUM_CORES + lax.axis_index("c")

    pltpu.sync_copy(ctx_idx.at[pl.ds(wid * B_PER_W * NW_CTX,
                                     B_PER_W * NW_CTX)], cidx)
    pltpu.sync_copy(tgt_idx.at[pl.ds(wid * B_PER_W * NT, B_PER_W * NT)], tidx)

    def c_slice(k, g):
        return cidx.at[pl.ds(k * CHUNK * NW_CTX + g * 128, 128)]

    def t_slice(k, g):
        return tidx.at[pl.ds(k * CHUNK * NT + g * TS, TS)]

    def gathers(k, crows, trows, sem):
        ops = []
        for g in range(G_CTX):
            ops.append(pltpu.make_async_copy(
                ctx_table.at[c_slice(k, g)],
                crows.at[pl.ds(g * 128, 128)], sem))
        for g in range(G_TGT):
            ops.append(pltpu.make_async_copy(
                tgt_table.at[t_slice(k, g)],
                trows.at[pl.ds(g * TS, TS)], sem))
        return ops

    def start_gathers(k, crows, trows, sem):
        for op in gathers(k, crows, trows, sem):
            op.start()

    def wait_gathers(k, crows, trows, sem):
        for op in gathers(k, crows, trows, sem):
            op.wait()

    start_gathers(0, crows_a, trows_a, sem_a)

    def chunk_body(k2, carry):
        bufs = ((crows_a, trows_a, sem_a), (crows_b, trows_b, sem_b))
        for half in range(2):
            k = k2 * 2 + half
            crows, trows, sem = bufs[half]
            ncr, ntr, nsem = bufs[1 - half]

            @pl.when(k + 1 < NCHUNK)
            def _():
                start_gathers(k + 1, ncr, ntr, nsem)
            wait_gathers(k, crows, trows, sem)
            compute_chunk(k, crows, trows)
        return carry

    def compute_chunk(k, crows, trows):

        last_lane = lax.iota(jnp.int32, HALF) == (HALF - 1)

        def elem(b, c2):
            r = b * NW_CTX
            s0 = crows[r, pl.ds(0, HALF)]
            s1 = crows[r, pl.ds(HALF, HALF)]
            for w in range(1, NW_CTX):
                s0 = s0 + crows[r + w, pl.ds(0, HALF)]
                s1 = s1 + crows[r + w, pl.ds(HALF, HALF)]
            t = b * NT
            for n in range(NT):
                q = (trows[t + n, pl.ds(0, HALF)] * s0
                     + trows[t + n, pl.ds(HALF, HALF)] * s1)
                c = plsc.cumsum(q)
                plsc.store_scatter(outv, [jnp.full((HALF,), t + n, jnp.int32)],
                                   c, mask=last_lane)
            return c2

        lax.fori_loop(0, CHUNK, elem, 0)
        pltpu.sync_copy(outv,
                        out.at[pl.ds(wid * (B_PER_W * NT) + k * CHUNK * NT,
                                     CHUNK * NT)])

    lax.fori_loop(0, NCHUNK // 2, chunk_body, 0)


def kernel(context, target, ctx_table, tgt_table):
    ctx_idx = context.astype(jnp.int32).reshape(-1)
    tgt_idx = target.astype(jnp.int32).reshape(-1)
    out = _cbow_sc(ctx_idx, tgt_idx, ctx_table, tgt_table)
    return out.reshape(BATCH, NT)
